# baseline (speedup 1.0000x reference)
import jax
import jax.numpy as jnp
from jax.experimental import pallas as pl
from jax.experimental.pallas import tpu as pltpu

CHUNK = 64
T_BLOCK = 1024


def _body(x_ref, o_ref):
    x = x_ref[...]
    pos = jax.lax.broadcasted_iota(jnp.int32, x.shape, 3) % CHUNK
    v = x
    for k in (1, 2, 4, 8, 16, 32):
        v = v + jnp.where(pos >= k, pltpu.roll(v, k, axis=3), 0.0)
    o_ref[...] = v


def kernel(g):
    B, T, H = g.shape
    gt = jnp.transpose(g, (0, 2, 1)).reshape(B, H, T // 128, 128)
    ot = pl.pallas_call(
        _body,
        out_shape=jax.ShapeDtypeStruct((B, H, T // 128, 128), jnp.float32),
        grid=(T // T_BLOCK,),
        in_specs=[pl.BlockSpec((B, H, T_BLOCK // 128, 128),
                               lambda i: (0, 0, i, 0))],
        out_specs=pl.BlockSpec((B, H, T_BLOCK // 128, 128),
                               lambda i: (0, 0, i, 0)),
    )(gt)
    return jnp.transpose(ot.reshape(B, H, T), (0, 2, 1))


# speedup vs baseline: 1.9966x; 1.9966x over previous
import jax
import jax.numpy as jnp
from jax.experimental import pallas as pl
from jax.experimental.pallas import tpu as pltpu

CHUNK = 64
T_BLOCK = 2048


def _body(x_ref, o_ref):
    x = x_ref[...]
    pos = jax.lax.broadcasted_iota(jnp.int32, x.shape, 2) % CHUNK
    v = x
    for k in (1, 2, 4, 8, 16, 32):
        v = v + jnp.where(pos >= k, pltpu.roll(v, k, axis=2), 0.0)
    o_ref[...] = v


def kernel(g):
    B, T, H = g.shape
    gt = jnp.transpose(g, (0, 2, 1))
    ot = pl.pallas_call(
        _body,
        out_shape=jax.ShapeDtypeStruct((B, H, T), jnp.float32),
        grid=(T // T_BLOCK,),
        in_specs=[pl.BlockSpec((B, H, T_BLOCK), lambda i: (0, 0, i))],
        out_specs=pl.BlockSpec((B, H, T_BLOCK), lambda i: (0, 0, i)),
    )(gt)
    return jnp.transpose(ot, (0, 2, 1))


# speedup vs baseline: 2.0020x; 1.0027x over previous
import jax
import jax.numpy as jnp
from jax.experimental import pallas as pl
from jax.experimental.pallas import tpu as pltpu

CHUNK = 64
T_BLOCK = 4096


def _body(x_ref, o_ref):
    x = x_ref[...]
    pos = jax.lax.broadcasted_iota(jnp.int32, x.shape, 2) % CHUNK
    v = x
    for k in (1, 2, 4, 8, 16, 32):
        v = v + jnp.where(pos >= k, pltpu.roll(v, k, axis=2), 0.0)
    o_ref[...] = v


def kernel(g):
    B, T, H = g.shape
    gt = jnp.transpose(g, (0, 2, 1))
    ot = pl.pallas_call(
        _body,
        out_shape=jax.ShapeDtypeStruct((B, H, T), jnp.float32),
        grid=(T // T_BLOCK,),
        in_specs=[pl.BlockSpec((B, H, T_BLOCK), lambda i: (0, 0, i))],
        out_specs=pl.BlockSpec((B, H, T_BLOCK), lambda i: (0, 0, i)),
    )(gt)
    return jnp.transpose(ot, (0, 2, 1))
